# baseline (speedup 1.0000x reference)
import functools

import jax
import jax.numpy as jnp
from jax.experimental import pallas as pl
from jax.experimental.pallas import tpu as pltpu

B, S, C, P = 16, 2048, 128, 256
TBLK = 128
RB = 2048


def _stageA_kernel(vecs_ref, w_ref, e_ref, m_ref, b1_ref, dec_ref, logit_ref):
    d = jnp.dot(vecs_ref[...], w_ref[...], preferred_element_type=jnp.float32)
    dec_ref[...] = d
    n2 = jnp.dot(d * d, m_ref[...], preferred_element_type=jnp.float32)
    dt = jnp.dot(d, e_ref[...], preferred_element_type=jnp.float32)
    a = dt * jax.lax.rsqrt(n2)
    logit_ref[...] = jnp.where(a > 0, a, jnp.exp(a) - 1.0) + b1_ref[0, 0]


def _stageB_kernel(logit_ref, nt_ref, mask_ref, eos_ref):
    l = logit_ref[...]
    rowmax = jnp.max(l, axis=1, keepdims=True)
    sumexp = jnp.sum(jnp.exp(l - rowmax), axis=1, keepdims=True)
    valid = jnp.logical_and(1.0 > 0.5 * sumexp,
                            jax.nn.sigmoid(rowmax) > 0.5)
    iota = jax.lax.broadcasted_iota(jnp.int32, (B, S), 1)
    idx = jnp.min(jnp.where(l == rowmax, iota, S), axis=1, keepdims=True)
    nt = jnp.where(valid, idx, S)
    nt_ref[...] = nt
    mask_ref[...] = (iota > nt).astype(jnp.int32)
    eos_ref[...] = (iota == nt).astype(jnp.int32)


def _stageC_kernel(dec_ref, mask_ref, wdec_ref, out_ref):
    post = jnp.dot(dec_ref[...], wdec_ref[...], preferred_element_type=jnp.float32)
    keep = (mask_ref[...] == 0).astype(jnp.float32)
    out_ref[...] = post * keep


@jax.jit
def kernel(vecs, W_decomp, W_dec, eos_vector, classifier1w, classifier1b):
    en = jnp.sqrt(jnp.sum(eos_vector * eos_vector))
    scale = jnp.abs(classifier1w[0]) / en
    blk = (jax.lax.broadcasted_iota(jnp.int32, (TBLK * C, TBLK), 0) // C ==
           jax.lax.broadcasted_iota(jnp.int32, (TBLK * C, TBLK), 1))
    Mh = blk.astype(jnp.float32)
    Eh = Mh * jnp.tile(eos_vector, TBLK)[:, None] * scale
    b1 = classifier1b.reshape(1, 1)

    nblk = S // TBLK
    dec, logits = pl.pallas_call(
        _stageA_kernel,
        grid=(nblk,),
        in_specs=[
            pl.BlockSpec((B, P), lambda i: (0, 0)),
            pl.BlockSpec((P, TBLK * C), lambda i: (0, i)),
            pl.BlockSpec((TBLK * C, TBLK), lambda i: (0, 0)),
            pl.BlockSpec((TBLK * C, TBLK), lambda i: (0, 0)),
            pl.BlockSpec((1, 1), lambda i: (0, 0), memory_space=pltpu.SMEM),
        ],
        out_specs=[
            pl.BlockSpec((B, TBLK * C), lambda i: (0, i)),
            pl.BlockSpec((B, TBLK), lambda i: (0, i)),
        ],
        out_shape=[
            jax.ShapeDtypeStruct((B, S * C), jnp.float32),
            jax.ShapeDtypeStruct((B, S), jnp.float32),
        ],
    )(vecs, W_decomp, Eh, Mh, b1)

    nt, mask, eos_pos = pl.pallas_call(
        _stageB_kernel,
        out_shape=[
            jax.ShapeDtypeStruct((B, 1), jnp.int32),
            jax.ShapeDtypeStruct((B, S), jnp.int32),
            jax.ShapeDtypeStruct((B, S), jnp.int32),
        ],
    )(logits)

    dec2 = dec.reshape(B * S, C)
    mask2 = mask.reshape(B * S, 1)
    post = pl.pallas_call(
        _stageC_kernel,
        grid=(B * S // RB,),
        in_specs=[
            pl.BlockSpec((RB, C), lambda i: (i, 0)),
            pl.BlockSpec((RB, 1), lambda i: (i, 0)),
            pl.BlockSpec((C, C), lambda i: (0, 0)),
        ],
        out_specs=pl.BlockSpec((RB, C), lambda i: (i, 0)),
        out_shape=jax.ShapeDtypeStruct((B * S, C), jnp.float32),
    )(dec2, mask2, W_dec)

    return (post.reshape(B, S, C), nt.reshape(B), mask, eos_pos)


# speedup vs baseline: 1.1540x; 1.1540x over previous
import functools

import jax
import jax.numpy as jnp
from jax.experimental import pallas as pl
from jax.experimental.pallas import tpu as pltpu

B, S, C, P = 16, 2048, 128, 256
TBLK = 128
RB = 2048


def _stageA_kernel(vecs_ref, w_ref, eos_ref, b1_ref, dec_ref, logit_ref):
    d = jnp.dot(vecs_ref[...], w_ref[...], preferred_element_type=jnp.float32)
    dec_ref[...] = d
    d3 = d.reshape(B, TBLK, C)
    n2 = jnp.sum(d3 * d3, axis=-1)
    dt = jnp.sum(d3 * eos_ref[...][None, :, :], axis=-1)
    a = dt * jax.lax.rsqrt(n2)
    logit_ref[...] = jnp.where(a > 0, a, jnp.exp(a) - 1.0) + b1_ref[0, 0]


def _stageB_kernel(logit_ref, nt_ref, mask_ref, eos_ref):
    l = logit_ref[...]
    rowmax = jnp.max(l, axis=1, keepdims=True)
    sumexp = jnp.sum(jnp.exp(l - rowmax), axis=1, keepdims=True)
    valid = jnp.logical_and(1.0 > 0.5 * sumexp,
                            jax.nn.sigmoid(rowmax) > 0.5)
    iota = jax.lax.broadcasted_iota(jnp.int32, (B, S), 1)
    idx = jnp.min(jnp.where(l == rowmax, iota, S), axis=1, keepdims=True)
    nt = jnp.where(valid, idx, S)
    nt_ref[...] = nt
    mask_ref[...] = (iota > nt).astype(jnp.int32)
    eos_ref[...] = (iota == nt).astype(jnp.int32)


def _stageC_kernel(dec_ref, mask_ref, wdec_ref, out_ref):
    post = jnp.dot(dec_ref[...], wdec_ref[...], preferred_element_type=jnp.float32)
    keep = (mask_ref[...] == 0).astype(jnp.float32)
    out_ref[...] = post * keep


@jax.jit
def kernel(vecs, W_decomp, W_dec, eos_vector, classifier1w, classifier1b):
    en = jnp.sqrt(jnp.sum(eos_vector * eos_vector))
    scale = jnp.abs(classifier1w[0]) / en
    eos_scaled = (eos_vector * scale).reshape(1, C)
    b1 = classifier1b.reshape(1, 1)

    nblk = S // TBLK
    dec, logits = pl.pallas_call(
        _stageA_kernel,
        grid=(nblk,),
        in_specs=[
            pl.BlockSpec((B, P), lambda i: (0, 0)),
            pl.BlockSpec((P, TBLK * C), lambda i: (0, i)),
            pl.BlockSpec((1, C), lambda i: (0, 0)),
            pl.BlockSpec((1, 1), lambda i: (0, 0), memory_space=pltpu.SMEM),
        ],
        out_specs=[
            pl.BlockSpec((B, TBLK * C), lambda i: (0, i)),
            pl.BlockSpec((B, TBLK), lambda i: (0, i)),
        ],
        out_shape=[
            jax.ShapeDtypeStruct((B, S * C), jnp.float32),
            jax.ShapeDtypeStruct((B, S), jnp.float32),
        ],
    )(vecs, W_decomp, eos_scaled, b1)

    nt, mask, eos_pos = pl.pallas_call(
        _stageB_kernel,
        out_shape=[
            jax.ShapeDtypeStruct((B, 1), jnp.int32),
            jax.ShapeDtypeStruct((B, S), jnp.int32),
            jax.ShapeDtypeStruct((B, S), jnp.int32),
        ],
    )(logits)

    dec2 = dec.reshape(B * S, C)
    mask2 = mask.reshape(B * S, 1)
    post = pl.pallas_call(
        _stageC_kernel,
        grid=(B * S // RB,),
        in_specs=[
            pl.BlockSpec((RB, C), lambda i: (i, 0)),
            pl.BlockSpec((RB, 1), lambda i: (i, 0)),
            pl.BlockSpec((C, C), lambda i: (0, 0)),
        ],
        out_specs=pl.BlockSpec((RB, C), lambda i: (i, 0)),
        out_shape=jax.ShapeDtypeStruct((B * S, C), jnp.float32),
    )(dec2, mask2, W_dec)

    return (post.reshape(B, S, C), nt.reshape(B), mask, eos_pos)


# speedup vs baseline: 1.7013x; 1.4743x over previous
import jax
import jax.numpy as jnp
from jax.experimental import pallas as pl
from jax.experimental.pallas import tpu as pltpu

B, S, C, P = 16, 2048, 128, 256
TBLK = 128
NBLK = S // TBLK


def _fused_kernel(vecs_ref, w_ref, eos_ref, b1_ref, wdec_ref,
                  post_hbm, nt_hbm, mask_hbm, eosp_hbm,
                  dec_s, log_s, nt_s, mask_s, eosp_s, obuf_s, sems):
    i = pl.program_id(0)
    d = jnp.dot(vecs_ref[...], w_ref[...], preferred_element_type=jnp.float32)
    d3 = d.reshape(B, TBLK, C)
    dec_s[:, pl.ds(i * TBLK, TBLK), :] = d3
    n2 = jnp.sum(d3 * d3, axis=-1)
    dt = jnp.sum(d3 * eos_ref[...][None], axis=-1)
    a = dt * jax.lax.rsqrt(n2)
    log_s[i] = jnp.where(a > 0, a, jnp.exp(a) - 1.0) + b1_ref[0, 0]

    @pl.when(i == NBLK - 1)
    def _phase2():
        l = log_s[...]
        rm = jnp.max(jnp.max(l, axis=2, keepdims=True), axis=0, keepdims=True)
        se = jnp.sum(jnp.sum(jnp.exp(l - rm), axis=2, keepdims=True),
                     axis=0, keepdims=True)
        valid = jnp.logical_and(se < 2.0, rm > 0.0)
        gi = (jax.lax.broadcasted_iota(jnp.int32, l.shape, 0) * TBLK +
              jax.lax.broadcasted_iota(jnp.int32, l.shape, 2))
        idx = jnp.min(jnp.min(jnp.where(l == rm, gi, S), axis=2, keepdims=True),
                      axis=0, keepdims=True)
        nt = jnp.where(valid, idx, S)
        ntc = nt.reshape(B, 1)
        nt_s[...] = ntc
        iota_s = jax.lax.broadcasted_iota(jnp.int32, (B, S), 1)
        mask_s[...] = (iota_s > ntc).astype(jnp.int32)
        eosp_s[...] = (iota_s == ntc).astype(jnp.int32)
        small = [
            pltpu.make_async_copy(nt_s, nt_hbm, sems.at[2]),
            pltpu.make_async_copy(mask_s, mask_hbm, sems.at[3]),
            pltpu.make_async_copy(eosp_s, eosp_hbm, sems.at[4]),
        ]
        for c in small:
            c.start()
        gi2 = jax.lax.broadcasted_iota(jnp.int32, (S, C), 0)
        wdec = wdec_ref[...]
        prev = [None, None]
        for b in range(B):
            ntb = ntc[b:b + 1, :]
            masked = jnp.where(gi2 > ntb, 0.0, dec_s[b])
            if prev[b % 2] is not None:
                prev[b % 2].wait()
            obuf_s[b % 2] = jnp.dot(masked, wdec,
                                    preferred_element_type=jnp.float32)
            cp = pltpu.make_async_copy(
                obuf_s.at[b % 2], post_hbm.at[pl.ds(b * S, S), :],
                sems.at[b % 2])
            cp.start()
            prev[b % 2] = cp
        for c in small:
            c.wait()
        prev[0].wait()
        prev[1].wait()


@jax.jit
def kernel(vecs, W_decomp, W_dec, eos_vector, classifier1w, classifier1b):
    en = jnp.sqrt(jnp.sum(eos_vector * eos_vector))
    scale = jnp.abs(classifier1w[0]) / en
    eos_scaled = (eos_vector * scale).reshape(1, C)
    b1 = classifier1b.reshape(1, 1)

    post, nt, mask, eos_pos = pl.pallas_call(
        _fused_kernel,
        grid=(NBLK,),
        in_specs=[
            pl.BlockSpec((B, P), lambda i: (0, 0)),
            pl.BlockSpec((P, TBLK * C), lambda i: (0, i)),
            pl.BlockSpec((1, C), lambda i: (0, 0)),
            pl.BlockSpec((1, 1), lambda i: (0, 0), memory_space=pltpu.SMEM),
            pl.BlockSpec((C, C), lambda i: (0, 0)),
        ],
        out_specs=[
            pl.BlockSpec(memory_space=pl.ANY),
            pl.BlockSpec(memory_space=pl.ANY),
            pl.BlockSpec(memory_space=pl.ANY),
            pl.BlockSpec(memory_space=pl.ANY),
        ],
        out_shape=[
            jax.ShapeDtypeStruct((B * S, C), jnp.float32),
            jax.ShapeDtypeStruct((B, 1), jnp.int32),
            jax.ShapeDtypeStruct((B, S), jnp.int32),
            jax.ShapeDtypeStruct((B, S), jnp.int32),
        ],
        scratch_shapes=[
            pltpu.VMEM((B, S, C), jnp.float32),
            pltpu.VMEM((NBLK, B, TBLK), jnp.float32),
            pltpu.VMEM((B, 1), jnp.int32),
            pltpu.VMEM((B, S), jnp.int32),
            pltpu.VMEM((B, S), jnp.int32),
            pltpu.VMEM((2, S, C), jnp.float32),
            pltpu.SemaphoreType.DMA((5,)),
        ],
        compiler_params=pltpu.CompilerParams(
            dimension_semantics=("arbitrary",),
        ),
    )(vecs, W_decomp, eos_scaled, b1, W_dec)

    return (post.reshape(B, S, C), nt.reshape(B), mask, eos_pos)


# speedup vs baseline: 1.8055x; 1.0612x over previous
import jax
import jax.numpy as jnp
from jax.experimental import pallas as pl
from jax.experimental.pallas import tpu as pltpu

B, S, C, P = 16, 2048, 128, 256
TBLK = 128
NBLK = S // TBLK


def _fused_kernel(vecs_ref, w_ref, eos_ref, b1_ref, wdec_ref,
                  post_hbm, nt_hbm, mask_hbm, eosp_hbm,
                  post_s, log_s, nt_s, mask_s, eosp_s, sems):
    i = pl.program_id(0)
    d = jnp.dot(vecs_ref[...], w_ref[...], preferred_element_type=jnp.float32)
    d3 = d.reshape(B, TBLK, C)
    n2 = jnp.sum(d3 * d3, axis=-1)
    dt = jnp.sum(d3 * eos_ref[...][None], axis=-1)
    a = dt * jax.lax.rsqrt(n2)
    log_s[i] = jnp.where(a > 0, a, jnp.exp(a) - 1.0) + b1_ref[0, 0]
    r = jnp.dot(d3.reshape(B * TBLK, C), wdec_ref[...],
                preferred_element_type=jnp.float32)
    post_s[:, pl.ds(i * TBLK, TBLK), :] = r.reshape(B, TBLK, C)

    @pl.when(i == NBLK - 1)
    def _phase2():
        l = log_s[...]
        rm = jnp.max(jnp.max(l, axis=2, keepdims=True), axis=0, keepdims=True)
        se = jnp.sum(jnp.sum(jnp.exp(l - rm), axis=2, keepdims=True),
                     axis=0, keepdims=True)
        valid = jnp.logical_and(se < 2.0, rm > 0.0)
        gi = (jax.lax.broadcasted_iota(jnp.int32, l.shape, 0) * TBLK +
              jax.lax.broadcasted_iota(jnp.int32, l.shape, 2))
        idx = jnp.min(jnp.min(jnp.where(l == rm, gi, S), axis=2, keepdims=True),
                      axis=0, keepdims=True)
        nt = jnp.where(valid, idx, S)
        ntc = nt.reshape(B, 1)
        nt_s[...] = ntc
        iota_s = jax.lax.broadcasted_iota(jnp.int32, (B, S), 1)
        mask_s[...] = (iota_s > ntc).astype(jnp.int32)
        eosp_s[...] = (iota_s == ntc).astype(jnp.int32)
        small = [
            pltpu.make_async_copy(nt_s, nt_hbm, sems.at[B]),
            pltpu.make_async_copy(mask_s, mask_hbm, sems.at[B + 1]),
            pltpu.make_async_copy(eosp_s, eosp_hbm, sems.at[B + 2]),
        ]
        for cp in small:
            cp.start()
        gi2 = jax.lax.broadcasted_iota(jnp.int32, (S, C), 0)
        big = []
        for b in range(B):
            ntb = ntc[b:b + 1, :]
            post_s[b] = jnp.where(gi2 > ntb, 0.0, post_s[b])
            cp = pltpu.make_async_copy(post_s.at[b], post_hbm.at[b],
                                       sems.at[b])
            cp.start()
            big.append(cp)
        for cp in small:
            cp.wait()
        for cp in big:
            cp.wait()


@jax.jit
def kernel(vecs, W_decomp, W_dec, eos_vector, classifier1w, classifier1b):
    en = jnp.sqrt(jnp.sum(eos_vector * eos_vector))
    scale = jnp.abs(classifier1w[0]) / en
    eos_scaled = (eos_vector * scale).reshape(1, C)
    b1 = classifier1b.reshape(1, 1)

    post, nt, mask, eos_pos = pl.pallas_call(
        _fused_kernel,
        grid=(NBLK,),
        in_specs=[
            pl.BlockSpec((B, P), lambda i: (0, 0)),
            pl.BlockSpec((P, TBLK * C), lambda i: (0, i)),
            pl.BlockSpec((1, C), lambda i: (0, 0)),
            pl.BlockSpec((1, 1), lambda i: (0, 0), memory_space=pltpu.SMEM),
            pl.BlockSpec((C, C), lambda i: (0, 0)),
        ],
        out_specs=[
            pl.BlockSpec(memory_space=pl.ANY),
            pl.BlockSpec(memory_space=pl.ANY),
            pl.BlockSpec(memory_space=pl.ANY),
            pl.BlockSpec(memory_space=pl.ANY),
        ],
        out_shape=[
            jax.ShapeDtypeStruct((B, S, C), jnp.float32),
            jax.ShapeDtypeStruct((B, 1), jnp.int32),
            jax.ShapeDtypeStruct((B, S), jnp.int32),
            jax.ShapeDtypeStruct((B, S), jnp.int32),
        ],
        scratch_shapes=[
            pltpu.VMEM((B, S, C), jnp.float32),
            pltpu.VMEM((NBLK, B, TBLK), jnp.float32),
            pltpu.VMEM((B, 1), jnp.int32),
            pltpu.VMEM((B, S), jnp.int32),
            pltpu.VMEM((B, S), jnp.int32),
            pltpu.SemaphoreType.DMA((B + 3,)),
        ],
        compiler_params=pltpu.CompilerParams(
            dimension_semantics=("arbitrary",),
        ),
    )(vecs, W_decomp, eos_scaled, b1, W_dec)

    return (post, nt.reshape(B), mask, eos_pos)
